# baseline (speedup 1.0000x reference)
import functools

import jax
import jax.numpy as jnp
from jax import lax
from jax.experimental import pallas as pl
from jax.experimental.pallas import tpu as pltpu
from jax.experimental.pallas import tpu_sc as plsc

N_NODES = 10000
N_EDGES = 320000
DIM = 128

NC, NS = 2, 16
NW = NC * NS
CH = 128
K0 = 144
K1 = 16
TOT_CH = NS * (K0 + K1)
NCH = TOT_CH // NW
NPAD = 10240
EPAD = TOT_CH * CH
ROWS_PER_TILE = NPAD // NS

_mesh = plsc.VectorSubcoreMesh(
    core_axis_name="c", subcore_axis_name="s", num_cores=NC, num_subcores=NS
)


@functools.partial(
    pl.kernel,
    out_type=jax.ShapeDtypeStruct((NC, NPAD), jnp.float32),
    mesh=_mesh,
    scratch_types=[
        pltpu.VMEM_SHARED((NPAD,), jnp.float32),
        pltpu.VMEM((NCH, CH), jnp.int32),
        pltpu.VMEM((CH,), jnp.float32),
    ],
)
def _degree_sc(dst_hbm, ones_hbm, zeros_hbm, out_hbm, hist_sp, didx, ones_v):
    c = lax.axis_index("c")
    s = lax.axis_index("s")
    w = s * NC + c
    pltpu.sync_copy(dst_hbm.at[w], didx)
    pltpu.sync_copy(ones_hbm, ones_v)
    pltpu.sync_copy(zeros_hbm, hist_sp.at[pl.ds(s * ROWS_PER_TILE, ROWS_PER_TILE)])
    plsc.subcore_barrier()

    def chunk(j, carry):
        pltpu.sync_copy(ones_v, hist_sp.at[didx.at[j]], add=True)
        return carry

    lax.fori_loop(0, NCH, chunk, 0)
    plsc.subcore_barrier()
    pltpu.sync_copy(
        hist_sp.at[pl.ds(s * ROWS_PER_TILE, ROWS_PER_TILE)],
        out_hbm.at[c, pl.ds(s * ROWS_PER_TILE, ROWS_PER_TILE)],
    )


@functools.partial(
    pl.kernel,
    out_type=jax.ShapeDtypeStruct((NC, NPAD, DIM), jnp.float32),
    mesh=_mesh,
    scratch_types=[
        pltpu.VMEM_SHARED((NPAD, DIM), jnp.float32),
        pltpu.VMEM((4, 2, CH), jnp.int32),
        pltpu.VMEM((CH,), jnp.int32),
        pltpu.VMEM((CH,), jnp.int32),
        pltpu.VMEM((CH, DIM), jnp.float32),
        pltpu.VMEM((CH, DIM), jnp.float32),
        pltpu.SemaphoreType.DMA,
        pltpu.SemaphoreType.DMA,
        pltpu.SemaphoreType.DMA,
        pltpu.SemaphoreType.DMA,
        pltpu.SemaphoreType.DMA,
        pltpu.SemaphoreType.DMA,
    ],
)
def _scatter_sc(g_hbm, e_hbm, src_hbm, dst_hbm, zeros_hbm, out_hbm,
                accum, ibuf, sidx1, didx1, rows0, rows1,
                gsem0, gsem1, isem0, isem1, isem2, isem3):
    c = lax.axis_index("c")
    s = lax.axis_index("s")
    rows = (rows0, rows1)
    gsems = (gsem0, gsem1)
    isems = (isem0, isem1, isem2, isem3)

    pltpu.sync_copy(zeros_hbm, accum.at[pl.ds(s * ROWS_PER_TILE, ROWS_PER_TILE)])
    plsc.subcore_barrier()

    def run_pipeline(base, K):
        def start_idx(j, q):
            pltpu.async_copy(e_hbm.at[base + j], ibuf.at[q], isems[q])

        def wait_idx(q):
            pltpu.make_async_copy(e_hbm.at[0], ibuf.at[q], isems[q]).wait()

        def start_gather(q, b):
            pltpu.async_copy(g_hbm.at[ibuf.at[q, 0]], rows[b], gsems[b])

        def wait_gather(b):
            pltpu.make_async_copy(g_hbm.at[pl.ds(0, CH)], rows[b], gsems[b]).wait()

        def scatter(q, b):
            pltpu.sync_copy(rows[b], accum.at[ibuf.at[q, 1]], add=True)

        for q in range(4):
            start_idx(q, q)
        for b in range(2):
            wait_idx(b)
            start_gather(b, b)

        def body(i, carry):
            for b4 in range(4):
                j = 4 * i + b4
                b = b4 % 2
                wait_gather(b)
                scatter(b4, b)
                start_idx(j + 4, b4)
                wait_idx((b4 + 2) % 4)
                start_gather((b4 + 2) % 4, b)
            return carry

        lax.fori_loop(0, (K - 8) // 4, body, 0)
        for b4 in range(8):
            j = K - 8 + b4
            q = j % 4
            b = j % 2
            wait_gather(b)
            scatter(q, b)
            if j + 4 < K:
                start_idx(j + 4, q)
            if j + 2 < K:
                wait_idx((q + 2) % 4)
                start_gather((q + 2) % 4, b)

    def run_sync(base, K):
        def chunk(j, carry):
            off = (base + j) * CH
            pltpu.sync_copy(src_hbm.at[pl.ds(off, CH)], sidx1)
            pltpu.sync_copy(dst_hbm.at[pl.ds(off, CH)], didx1)
            pltpu.async_copy(g_hbm.at[sidx1], rows0, gsem0, priority=1).wait()
            pltpu.async_copy(rows0, accum.at[didx1], gsem1, add=True,
                             priority=1).wait()
            return carry

        lax.fori_loop(0, K, chunk, 0)

    @pl.when(c == 0)
    def _pipe0():
        run_pipeline(s * K0, K0)

    @pl.when(c == 1)
    def _pipe1():
        run_sync(NS * K0 + s * K1, K1)

    plsc.subcore_barrier()
    pltpu.sync_copy(
        accum.at[pl.ds(s * ROWS_PER_TILE, ROWS_PER_TILE)],
        out_hbm.at[c, pl.ds(s * ROWS_PER_TILE, ROWS_PER_TILE)],
    )


_BLK = 1024
_GRID = NPAD // _BLK


def _row_spec():
    return pl.BlockSpec((_BLK, DIM), lambda i: (i, 0))


def _col_spec():
    return pl.BlockSpec((_BLK, 1), lambda i: (i, 0))


def _w_spec():
    return pl.BlockSpec((DIM, DIM), lambda i: (0, 0))


def _first_body(x_ref, w_ref, h0_ref, h1_ref, g_ref, dinv_ref):
    dinv = lax.rsqrt(1.0 + h0_ref[...] + h1_ref[...])
    dinv_ref[...] = dinv
    g_ref[...] = jnp.dot(x_ref[...], w_ref[...],
                         preferred_element_type=jnp.float32) * dinv


_first_tc = pl.pallas_call(
    _first_body,
    grid=(_GRID,),
    in_specs=[_row_spec(), _w_spec(), _col_spec(), _col_spec()],
    out_specs=[_row_spec(), _col_spec()],
    out_shape=[
        jax.ShapeDtypeStruct((NPAD, DIM), jnp.float32),
        jax.ShapeDtypeStruct((NPAD, 1), jnp.float32),
    ],
)


def _mid_body(p0_ref, p1_ref, g_ref, dinv_ref, w_ref, o_ref):
    dinv = dinv_ref[...]
    h = jnp.maximum((p0_ref[...] + p1_ref[...] + g_ref[...]) * dinv, 0.0)
    o_ref[...] = jnp.dot(h, w_ref[...],
                         preferred_element_type=jnp.float32) * dinv


_mid_tc = pl.pallas_call(
    _mid_body,
    grid=(_GRID,),
    in_specs=[_row_spec(), _row_spec(), _row_spec(), _col_spec(), _w_spec()],
    out_specs=_row_spec(),
    out_shape=jax.ShapeDtypeStruct((NPAD, DIM), jnp.float32),
)


def _last_body(p0_ref, p1_ref, g_ref, dinv_ref, o_ref):
    h = jnp.maximum((p0_ref[...] + p1_ref[...] + g_ref[...]) * dinv_ref[...], 0.0)
    nrm = jnp.sqrt(jnp.sum(h * h, axis=1, keepdims=True))
    o_ref[...] = h / jnp.maximum(nrm, 1e-12)


_last_tc = pl.pallas_call(
    _last_body,
    grid=(_GRID,),
    in_specs=[_row_spec(), _row_spec(), _row_spec(), _col_spec()],
    out_specs=_row_spec(),
    out_shape=jax.ShapeDtypeStruct((NPAD, DIM), jnp.float32),
)


def kernel(x, edge_index, W0, W1, W2):
    src = edge_index[0].astype(jnp.int32)
    dst = edge_index[1].astype(jnp.int32)
    pad_idx = jnp.full((EPAD - N_EDGES,), NPAD - 1, dtype=jnp.int32)
    src_f = jnp.concatenate([src, pad_idx]).reshape(TOT_CH, CH)
    dst_f = jnp.concatenate([dst, pad_idx]).reshape(TOT_CH, CH)
    dst_p = dst_f.reshape(NW, NCH, CH)
    e_p = jnp.stack([src_f, dst_f], axis=1)
    x_p = jnp.pad(x, ((0, NPAD - N_NODES), (0, 0)))

    ones_ch = jnp.ones((CH,), jnp.float32)
    zeros_row = jnp.zeros((ROWS_PER_TILE,), jnp.float32)
    zeros_blk = jnp.zeros((ROWS_PER_TILE, DIM), jnp.float32)

    hist = _degree_sc(dst_p, ones_ch, zeros_row)
    g, dinv = _first_tc(x_p, W0, hist[0][:, None], hist[1][:, None])
    for W in (W1, W2, None):
        p = _scatter_sc(g, e_p, src_f.reshape(EPAD), dst_f.reshape(EPAD), zeros_blk)
        if W is None:
            out = _last_tc(p[0], p[1], g, dinv)
        else:
            g = _mid_tc(p[0], p[1], g, dinv, W)
    return out[:N_NODES]


# speedup vs baseline: 2.7739x; 2.7739x over previous
import functools

import jax
import jax.numpy as jnp
from jax import lax
from jax.experimental import pallas as pl
from jax.experimental.pallas import tpu as pltpu
from jax.experimental.pallas import tpu_sc as plsc

N_NODES = 10000
N_EDGES = 320000
DIM = 128

NC, NS = 2, 16
NW = NC * NS
CH = 128
K0 = 80
K1 = 80
TOT_CH = NS * (K0 + K1)
NCH = TOT_CH // NW
NPAD = 10240
EPAD = TOT_CH * CH
ROWS_PER_TILE = NPAD // NS

_mesh = plsc.VectorSubcoreMesh(
    core_axis_name="c", subcore_axis_name="s", num_cores=NC, num_subcores=NS
)


@functools.partial(
    pl.kernel,
    out_type=jax.ShapeDtypeStruct((NC, NPAD), jnp.float32),
    mesh=_mesh,
    scratch_types=[
        pltpu.VMEM_SHARED((NPAD,), jnp.float32),
        pltpu.VMEM((NCH, CH), jnp.int32),
        pltpu.VMEM((CH,), jnp.float32),
    ],
)
def _degree_sc(dst_hbm, ones_hbm, zeros_hbm, out_hbm, hist_sp, didx, ones_v):
    c = lax.axis_index("c")
    s = lax.axis_index("s")
    w = s * NC + c
    pltpu.sync_copy(dst_hbm.at[w], didx)
    pltpu.sync_copy(ones_hbm, ones_v)
    pltpu.sync_copy(zeros_hbm, hist_sp.at[pl.ds(s * ROWS_PER_TILE, ROWS_PER_TILE)])
    plsc.subcore_barrier()

    def chunk(j, carry):
        pltpu.sync_copy(ones_v, hist_sp.at[didx.at[j]], add=True)
        return carry

    lax.fori_loop(0, NCH, chunk, 0)
    plsc.subcore_barrier()
    pltpu.sync_copy(
        hist_sp.at[pl.ds(s * ROWS_PER_TILE, ROWS_PER_TILE)],
        out_hbm.at[c, pl.ds(s * ROWS_PER_TILE, ROWS_PER_TILE)],
    )


@functools.partial(
    pl.kernel,
    out_type=jax.ShapeDtypeStruct((NC, NPAD, DIM), jnp.float32),
    mesh=_mesh,
    scratch_types=[
        pltpu.VMEM_SHARED((NPAD, DIM), jnp.float32),
        pltpu.VMEM((4, 2, CH), jnp.int32),
        pltpu.VMEM((CH,), jnp.int32),
        pltpu.VMEM((CH,), jnp.int32),
        pltpu.VMEM((CH, DIM), jnp.float32),
        pltpu.VMEM((CH, DIM), jnp.float32),
        pltpu.SemaphoreType.DMA,
        pltpu.SemaphoreType.DMA,
        pltpu.SemaphoreType.DMA,
        pltpu.SemaphoreType.DMA,
        pltpu.SemaphoreType.DMA,
        pltpu.SemaphoreType.DMA,
    ],
)
def _scatter_sc(g_hbm, e_hbm, src_hbm, dst_hbm, zeros_hbm, out_hbm,
                accum, ibuf, sidx1, didx1, rows0, rows1,
                gsem0, gsem1, isem0, isem1, isem2, isem3):
    c = lax.axis_index("c")
    s = lax.axis_index("s")
    rows = (rows0, rows1)
    gsems = (gsem0, gsem1)
    isems = (isem0, isem1, isem2, isem3)

    pltpu.sync_copy(zeros_hbm, accum.at[pl.ds(s * ROWS_PER_TILE, ROWS_PER_TILE)])
    plsc.subcore_barrier()

    def run_pipeline(base, K):
        def start_idx(j, q):
            pltpu.async_copy(e_hbm.at[base + j], ibuf.at[q], isems[q])

        def wait_idx(q):
            pltpu.make_async_copy(e_hbm.at[0], ibuf.at[q], isems[q]).wait()

        def start_gather(q, b):
            pltpu.async_copy(g_hbm.at[ibuf.at[q, 0]], rows[b], gsems[b])

        def wait_gather(b):
            pltpu.make_async_copy(g_hbm.at[pl.ds(0, CH)], rows[b], gsems[b]).wait()

        def scatter(q, b):
            pltpu.sync_copy(rows[b], accum.at[ibuf.at[q, 1]], add=True)

        for q in range(4):
            start_idx(q, q)
        for b in range(2):
            wait_idx(b)
            start_gather(b, b)

        def body(i, carry):
            for b4 in range(4):
                j = 4 * i + b4
                b = b4 % 2
                wait_gather(b)
                scatter(b4, b)
                start_idx(j + 4, b4)
                wait_idx((b4 + 2) % 4)
                start_gather((b4 + 2) % 4, b)
            return carry

        lax.fori_loop(0, (K - 8) // 4, body, 0)
        for b4 in range(8):
            j = K - 8 + b4
            q = j % 4
            b = j % 2
            wait_gather(b)
            scatter(q, b)
            if j + 4 < K:
                start_idx(j + 4, q)
            if j + 2 < K:
                wait_idx((q + 2) % 4)
                start_gather((q + 2) % 4, b)

    def run_sync(base, K):
        def chunk(j, carry):
            off = (base + j) * CH
            pltpu.sync_copy(src_hbm.at[pl.ds(off, CH)], sidx1)
            pltpu.sync_copy(dst_hbm.at[pl.ds(off, CH)], didx1)
            pltpu.async_copy(g_hbm.at[sidx1], rows0, gsem0, priority=1).wait()
            pltpu.async_copy(rows0, accum.at[didx1], gsem1, add=True,
                             priority=1).wait()
            return carry

        lax.fori_loop(0, K, chunk, 0)

    @pl.when(c == 0)
    def _pipe0():
        run_pipeline(s * K0, K0)

    @pl.when(c == 1)
    def _pipe1():
        run_pipeline(NS * K0 + s * K1, K1)

    plsc.subcore_barrier()
    pltpu.sync_copy(
        accum.at[pl.ds(s * ROWS_PER_TILE, ROWS_PER_TILE)],
        out_hbm.at[c, pl.ds(s * ROWS_PER_TILE, ROWS_PER_TILE)],
    )


_BLK = 1024
_GRID = NPAD // _BLK


def _row_spec():
    return pl.BlockSpec((_BLK, DIM), lambda i: (i, 0))


def _col_spec():
    return pl.BlockSpec((_BLK, 1), lambda i: (i, 0))


def _w_spec():
    return pl.BlockSpec((DIM, DIM), lambda i: (0, 0))


def _first_body(x_ref, w_ref, h0_ref, h1_ref, g_ref, dinv_ref):
    dinv = lax.rsqrt(1.0 + h0_ref[...] + h1_ref[...])
    dinv_ref[...] = dinv
    g_ref[...] = jnp.dot(x_ref[...], w_ref[...],
                         preferred_element_type=jnp.float32) * dinv


_first_tc = pl.pallas_call(
    _first_body,
    grid=(_GRID,),
    in_specs=[_row_spec(), _w_spec(), _col_spec(), _col_spec()],
    out_specs=[_row_spec(), _col_spec()],
    out_shape=[
        jax.ShapeDtypeStruct((NPAD, DIM), jnp.float32),
        jax.ShapeDtypeStruct((NPAD, 1), jnp.float32),
    ],
)


def _mid_body(p0_ref, p1_ref, g_ref, dinv_ref, w_ref, o_ref):
    dinv = dinv_ref[...]
    h = jnp.maximum((p0_ref[...] + p1_ref[...] + g_ref[...]) * dinv, 0.0)
    o_ref[...] = jnp.dot(h, w_ref[...],
                         preferred_element_type=jnp.float32) * dinv


_mid_tc = pl.pallas_call(
    _mid_body,
    grid=(_GRID,),
    in_specs=[_row_spec(), _row_spec(), _row_spec(), _col_spec(), _w_spec()],
    out_specs=_row_spec(),
    out_shape=jax.ShapeDtypeStruct((NPAD, DIM), jnp.float32),
)


def _last_body(p0_ref, p1_ref, g_ref, dinv_ref, o_ref):
    h = jnp.maximum((p0_ref[...] + p1_ref[...] + g_ref[...]) * dinv_ref[...], 0.0)
    nrm = jnp.sqrt(jnp.sum(h * h, axis=1, keepdims=True))
    o_ref[...] = h / jnp.maximum(nrm, 1e-12)


_last_tc = pl.pallas_call(
    _last_body,
    grid=(_GRID,),
    in_specs=[_row_spec(), _row_spec(), _row_spec(), _col_spec()],
    out_specs=_row_spec(),
    out_shape=jax.ShapeDtypeStruct((NPAD, DIM), jnp.float32),
)


def kernel(x, edge_index, W0, W1, W2):
    src = edge_index[0].astype(jnp.int32)
    dst = edge_index[1].astype(jnp.int32)
    pad_idx = N_NODES + jnp.arange(EPAD - N_EDGES, dtype=jnp.int32) % (NPAD - N_NODES)
    src_f = jnp.concatenate([src, pad_idx]).reshape(TOT_CH, CH)
    dst_f = jnp.concatenate([dst, pad_idx]).reshape(TOT_CH, CH)
    dst_p = dst_f.reshape(NW, NCH, CH)
    e_p = jnp.stack([src_f, dst_f], axis=1)
    x_p = jnp.pad(x, ((0, NPAD - N_NODES), (0, 0)))

    ones_ch = jnp.ones((CH,), jnp.float32)
    zeros_row = jnp.zeros((ROWS_PER_TILE,), jnp.float32)
    zeros_blk = jnp.zeros((ROWS_PER_TILE, DIM), jnp.float32)

    hist = _degree_sc(dst_p, ones_ch, zeros_row)
    g, dinv = _first_tc(x_p, W0, hist[0][:, None], hist[1][:, None])
    for W in (W1, W2, None):
        p = _scatter_sc(g, e_p, src_f.reshape(EPAD), dst_f.reshape(EPAD), zeros_blk)
        if W is None:
            out = _last_tc(p[0], p[1], g, dinv)
        else:
            g = _mid_tc(p[0], p[1], g, dinv, W)
    return out[:N_NODES]
